# Initial kernel scaffold; baseline (speedup 1.0000x reference)
import functools

import jax
import jax.numpy as jnp
from jax import lax
from jax.experimental import pallas as pl
from jax.experimental.pallas import tpu as pltpu
from jax.experimental.pallas import tpu_sc as plsc

E = 320000
R = 10000
D = 128
NC = 2
NS = 16
NWORK = NC * NS
EPT = E // NWORK
W = 125
NWIN = EPT // W
RPT = R // NS
LANES = 16


def _sc_scatter_body(i1_hbm, idx_hbm, out_hbm, idx_v, upd_v, zrow_v, acc_sh):
    c = lax.axis_index("c")
    s = lax.axis_index("s")
    wid = c * NS + s

    def zero_row(i, carry):
        for blk in range(D // LANES):
            zrow_v[i, pl.ds(blk * LANES, LANES)] = jnp.zeros((LANES,), jnp.float32)
        return carry

    lax.fori_loop(0, W, zero_row, 0)
    for r in range(RPT // W):
        pltpu.sync_copy(zrow_v, acc_sh.at[pl.ds(s * RPT + r * W, W)])
    plsc.subcore_barrier()

    pltpu.sync_copy(idx_hbm.at[wid], idx_v)

    ebase = wid * EPT

    def window(j, carry):
        pltpu.sync_copy(i1_hbm.at[pl.ds(ebase + j * W, W)], upd_v)
        pltpu.sync_copy(upd_v, acc_sh.at[idx_v.at[j]], add=True)
        return carry

    lax.fori_loop(0, NWIN, window, 0)
    plsc.subcore_barrier()

    rbase = s * RPT
    pltpu.sync_copy(
        acc_sh.at[pl.ds(rbase, RPT)],
        out_hbm.at[pl.ds(c * R + rbase, RPT)],
    )


_sc_scatter = functools.partial(
    pl.kernel,
    out_type=jax.ShapeDtypeStruct((NC * R, D), jnp.float32),
    mesh=plsc.VectorSubcoreMesh(
        core_axis_name="c", subcore_axis_name="s", num_cores=NC, num_subcores=NS
    ),
    scratch_types=[
        pltpu.VMEM((NWIN, W), jnp.int32),
        pltpu.VMEM((W, D), jnp.float32),
        pltpu.VMEM((W, D), jnp.float32),
        pltpu.VMEM_SHARED((R, D), jnp.float32),
    ],
)(_sc_scatter_body)


def _sum_partials_body(a_ref, b_ref, o_ref):
    o_ref[...] = a_ref[...] + b_ref[...]


def kernel(i1, pair_i, p1):
    del p1
    idx = pair_i.astype(jnp.int32).reshape(NWORK, NWIN, W)
    partials = _sc_scatter(i1, idx)
    blk = 1250
    out = pl.pallas_call(
        _sum_partials_body,
        out_shape=jax.ShapeDtypeStruct((R, D), jnp.float32),
        grid=(R // blk,),
        in_specs=[
            pl.BlockSpec((blk, D), lambda i: (i, 0)),
            pl.BlockSpec((blk, D), lambda i: (i, 0)),
        ],
        out_specs=pl.BlockSpec((blk, D), lambda i: (i, 0)),
    )(partials[:R], partials[R:])
    return out


# speedup vs baseline: 4.6152x; 4.6152x over previous
import functools

import jax
import jax.numpy as jnp
from jax import lax
from jax.experimental import pallas as pl
from jax.experimental.pallas import tpu as pltpu
from jax.experimental.pallas import tpu_sc as plsc

E = 320000
R = 10000
RP = 10240
D = 128
NC = 2
NS = 16
NWORK = NC * NS
EPT = E // NWORK
W = 80
NWIN = EPT // W
RPT = RP // NS
LANES = 16


def _sc_scatter_body(i1_hbm, idx_hbm, out_hbm, idx_v, upd_v, zrow_v, acc_sh):
    c = lax.axis_index("c")
    s = lax.axis_index("s")
    wid = c * NS + s

    def zero_row(i, carry):
        for blk in range(D // LANES):
            zrow_v[i, pl.ds(blk * LANES, LANES)] = jnp.zeros((LANES,), jnp.float32)
        return carry

    lax.fori_loop(0, W, zero_row, 0)
    for r in range(RPT // W):
        pltpu.sync_copy(zrow_v, acc_sh.at[pl.ds(s * RPT + r * W, W)])
    plsc.subcore_barrier()

    pltpu.sync_copy(idx_hbm.at[wid], idx_v)

    ebase = wid * EPT

    def window(j, carry):
        pltpu.sync_copy(i1_hbm.at[pl.ds(ebase + j * W, W)], upd_v)
        pltpu.sync_copy(upd_v, acc_sh.at[idx_v.at[j]], add=True)
        return carry

    lax.fori_loop(0, NWIN, window, 0)
    plsc.subcore_barrier()

    rbase = s * RPT
    pltpu.sync_copy(
        acc_sh.at[pl.ds(rbase, RPT)],
        out_hbm.at[c, pl.ds(rbase, RPT)],
    )


_sc_scatter = functools.partial(
    pl.kernel,
    out_type=jax.ShapeDtypeStruct((NC, RP, D), jnp.float32),
    mesh=plsc.VectorSubcoreMesh(
        core_axis_name="c", subcore_axis_name="s", num_cores=NC, num_subcores=NS
    ),
    scratch_types=[
        pltpu.VMEM((NWIN, W), jnp.int32),
        pltpu.VMEM((W, D), jnp.float32),
        pltpu.VMEM((W, D), jnp.float32),
        pltpu.VMEM_SHARED((RP, D), jnp.float32),
    ],
)(_sc_scatter_body)


def _sum_partials_body(a_ref, b_ref, o_ref):
    o_ref[...] = a_ref[0] + b_ref[0]


def kernel(i1, pair_i, p1):
    del p1
    idx = pair_i.astype(jnp.int32).reshape(NWORK, NWIN, W)
    partials = _sc_scatter(i1, idx)
    blk = 1000
    out = pl.pallas_call(
        _sum_partials_body,
        out_shape=jax.ShapeDtypeStruct((R, D), jnp.float32),
        grid=(R // blk,),
        in_specs=[
            pl.BlockSpec((1, blk, D), lambda i: (0, i, 0)),
            pl.BlockSpec((1, blk, D), lambda i: (1, i, 0)),
        ],
        out_specs=pl.BlockSpec((blk, D), lambda i: (i, 0)),
    )(partials, partials)
    return out


# speedup vs baseline: 7.4867x; 1.6222x over previous
import functools

import jax
import jax.numpy as jnp
from jax import lax
from jax.experimental import pallas as pl
from jax.experimental.pallas import tpu as pltpu
from jax.experimental.pallas import tpu_sc as plsc

E = 320000
R = 10000
RP = 10240
D = 128
NC = 2
NS = 16
NWORK = NC * NS
EPT = E // NWORK
W = 80
NWIN = EPT // W
RPT = RP // NS
LANES = 16
ZR = 16
NB = 2


def _sc_scatter_body(
    i1_hbm, idx_hbm, out_hbm, idx_v, upd_v, zrow_v, acc_sh, isem, gsems
):
    c = lax.axis_index("c")
    s = lax.axis_index("s")
    wid = c * NS + s
    ebase = wid * EPT

    idx_cp = pltpu.async_copy(idx_hbm.at[wid], idx_v, isem)
    prime = [
        pltpu.async_copy(
            i1_hbm.at[pl.ds(ebase + b * W, W)], upd_v.at[b], gsems[b]
        )
        for b in range(NB)
    ]

    def zero_row(i, carry):
        for blk in range(D // LANES):
            zrow_v[i, pl.ds(blk * LANES, LANES)] = jnp.zeros((LANES,), jnp.float32)
        return carry

    lax.fori_loop(0, ZR, zero_row, 0)
    for r in range(RPT // ZR):
        pltpu.sync_copy(zrow_v, acc_sh.at[pl.ds(s * RPT + r * ZR, ZR)])
    idx_cp.wait()
    plsc.subcore_barrier()

    def group(g, carry):
        for b in range(NB):
            j = g * NB + b
            prime[b].wait()
            pltpu.sync_copy(upd_v.at[b], acc_sh.at[idx_v.at[j]], add=True)
            pltpu.async_copy(
                i1_hbm.at[pl.ds(ebase + (j + NB) * W, W)], upd_v.at[b], gsems[b]
            )
        return carry

    lax.fori_loop(0, (NWIN - 1) // NB - 1, group, 0)
    base = ((NWIN - 1) // NB - 1) * NB
    for j in range(base, NWIN - 1):
        b = j % NB
        prime[b].wait()
        pltpu.sync_copy(upd_v.at[b], acc_sh.at[idx_v.at[j]], add=True)
    pltpu.sync_copy(i1_hbm.at[pl.ds(ebase + (NWIN - 1) * W, W)], upd_v.at[0])
    pltpu.sync_copy(upd_v.at[0], acc_sh.at[idx_v.at[NWIN - 1]], add=True)
    plsc.subcore_barrier()

    rbase = s * RPT
    pltpu.sync_copy(
        acc_sh.at[pl.ds(rbase, RPT)],
        out_hbm.at[c, pl.ds(rbase, RPT)],
    )


_sc_scatter = functools.partial(
    pl.kernel,
    out_type=jax.ShapeDtypeStruct((NC, RP, D), jnp.float32),
    mesh=plsc.VectorSubcoreMesh(
        core_axis_name="c", subcore_axis_name="s", num_cores=NC, num_subcores=NS
    ),
    scratch_types=[
        pltpu.VMEM((NWIN, W), jnp.int32),
        pltpu.VMEM((NB, W, D), jnp.float32),
        pltpu.VMEM((ZR, D), jnp.float32),
        pltpu.VMEM_SHARED((RP, D), jnp.float32),
        pltpu.SemaphoreType.DMA,
        [pltpu.SemaphoreType.DMA] * NB,
    ],
)(_sc_scatter_body)


def _sum_partials_body(a_ref, b_ref, o_ref):
    o_ref[...] = a_ref[0] + b_ref[0]


def kernel(i1, pair_i, p1):
    del p1
    idx = pair_i.astype(jnp.int32).reshape(NWORK, NWIN, W)
    partials = _sc_scatter(i1, idx)
    blk = 1000
    out = pl.pallas_call(
        _sum_partials_body,
        out_shape=jax.ShapeDtypeStruct((R, D), jnp.float32),
        grid=(R // blk,),
        in_specs=[
            pl.BlockSpec((1, blk, D), lambda i: (0, i, 0)),
            pl.BlockSpec((1, blk, D), lambda i: (1, i, 0)),
        ],
        out_specs=pl.BlockSpec((blk, D), lambda i: (i, 0)),
    )(partials, partials)
    return out
